# baseline (speedup 1.0000x reference)
import functools

import jax
import jax.numpy as jnp
from jax import lax
from jax.experimental import pallas as pl
from jax.experimental.pallas import tpu as pltpu
from jax.experimental.pallas import tpu_sc as plsc

BATCH = 16384
EMBED_DIM = 64
IDX_SEG = 128


@functools.cache
def _build():
    info = plsc.get_sparse_core_info()
    NC, NS, L = info.num_cores, info.num_subcores, info.num_lanes
    NW = NC * NS
    b_per_w = BATCH // NW
    n_seg = b_per_w // IDX_SEG
    mesh = plsc.VectorSubcoreMesh(core_axis_name="c", subcore_axis_name="s")

    @functools.partial(
        pl.kernel,
        mesh=mesh,
        out_type=jax.ShapeDtypeStruct((BATCH,), jnp.float32),
        compiler_params=pltpu.CompilerParams(
            needs_layout_passes=False, use_tc_tiling_on_sc=False),
        scratch_types=[
            pltpu.VMEM((n_seg, IDX_SEG), jnp.int32),
            pltpu.VMEM((n_seg, IDX_SEG), jnp.int32),
            pltpu.VMEM((b_per_w, EMBED_DIM), jnp.float32),
            pltpu.VMEM((b_per_w, EMBED_DIM), jnp.float32),
            pltpu.VMEM((b_per_w,), jnp.float32),
            pltpu.SemaphoreType.DMA,
        ],
    )
    def bprmf(u_ids_hbm, i_ids_hbm, user_t_hbm, item_t_hbm, out_hbm,
              uidx_v, iidx_v, urows_v, irows_v, out_v, sem):
        wid = lax.axis_index("s") * NC + lax.axis_index("c")
        base = wid * b_per_w

        pltpu.sync_copy(u_ids_hbm.at[pl.ds(wid * n_seg, n_seg)], uidx_v)
        pltpu.sync_copy(i_ids_hbm.at[pl.ds(wid * n_seg, n_seg)], iidx_v)

        copies = []
        for j in range(n_seg):
            copies.append(pltpu.async_copy(
                user_t_hbm.at[uidx_v.at[j]],
                urows_v.at[pl.ds(j * IDX_SEG, IDX_SEG)], sem))
            copies.append(pltpu.async_copy(
                item_t_hbm.at[iidx_v.at[j]],
                irows_v.at[pl.ds(j * IDX_SEG, IDX_SEG)], sem))
        for c in copies:
            c.wait()

        iota = lax.iota(jnp.int32, L)

        def group(g, carry):
            r_vec = g * L + iota
            acc = jnp.zeros((L,), jnp.float32)
            for d in range(EMBED_DIM):
                col = jnp.full((L,), d, jnp.int32)
                u = plsc.load_gather(urows_v, [r_vec, col])
                v = plsc.load_gather(irows_v, [r_vec, col])
                acc = acc + u * v
            out_v[pl.ds(g * L, L)] = acc
            return carry

        lax.fori_loop(0, b_per_w // L, group, 0)
        pltpu.sync_copy(out_v, out_hbm.at[pl.ds(base, b_per_w)])

    return bprmf


def kernel(u_ids, i_ids, user_table, item_table):
    u2 = u_ids.astype(jnp.int32).reshape(BATCH // IDX_SEG, IDX_SEG)
    i2 = i_ids.astype(jnp.int32).reshape(BATCH // IDX_SEG, IDX_SEG)
    return _build()(u2, i2, user_table, item_table)


# speedup vs baseline: 2.7581x; 2.7581x over previous
import functools

import jax
import jax.numpy as jnp
from jax import lax
from jax.experimental import pallas as pl
from jax.experimental.pallas import tpu as pltpu
from jax.experimental.pallas import tpu_sc as plsc

BATCH = 16384
D = 64
L = 16
NBUF = 8
H = 256


@functools.cache
def _build():
    info = plsc.get_sparse_core_info()
    NC = info.num_cores
    NW = NC * info.num_subcores
    b_per_w = BATCH // NW
    n_half = b_per_w // H
    ng = H // L
    mesh = plsc.VectorSubcoreMesh(core_axis_name="c", subcore_axis_name="s")

    @functools.partial(
        pl.kernel,
        mesh=mesh,
        out_type=jax.ShapeDtypeStruct((BATCH,), jnp.float32),
        compiler_params=pltpu.CompilerParams(
            needs_layout_passes=False, use_tc_tiling_on_sc=True),
        scratch_types=(
            [pltpu.VMEM((b_per_w,), jnp.int32)] * 2
            + [pltpu.VMEM((64, 128), jnp.float32)] * NBUF
            + [pltpu.VMEM((H * D,), jnp.float32)] * 2
            + [pltpu.VMEM((b_per_w,), jnp.float32)]
            + [pltpu.SemaphoreType.DMA] * NBUF
        ),
    )
    def bprmf(u_ids_hbm, i_ids_hbm, ut_hbm, it_hbm, out_hbm,
              uidv, iidv, *rest):
        bufs = rest[:NBUF]
        urows, irows, outv = rest[NBUF:NBUF + 3]
        sems = rest[NBUF + 3:]

        wid = lax.axis_index("s") * NC + lax.axis_index("c")
        base = wid * b_per_w
        pltpu.sync_copy(u_ids_hbm.at[pl.ds(base, b_per_w)], uidv)
        pltpu.sync_copy(i_ids_hbm.at[pl.ds(base, b_per_w)], iidv)

        iota = lax.iota(jnp.int32, L)
        rowvecs = [j * L + iota for j in range(4)]

        def fire(tab, slot, uid):
            col0 = pl.multiple_of(jnp.bitwise_and(uid, -128), 128)
            pltpu.async_copy(tab.at[:, pl.ds(col0, 128)], bufs[slot], sems[slot])

        def drain(tab, slot):
            pltpu.make_async_copy(
                tab.at[:, pl.ds(0, 128)], bufs[slot], sems[slot]).wait()

        def extract(slot, uid, rows, bglobal):
            col = jnp.full((L,), jnp.bitwise_and(uid, 127), jnp.int32)
            for j in range(4):
                v = plsc.load_gather(bufs[slot], [rowvecs[j], col])
                rows[pl.ds(bglobal * D + j * L, L)] = v

        def fetch_phase(tab, idv, rows, half):
            off = half * H
            first = idv[pl.ds(off, L)]
            for k in range(NBUF):
                fire(tab, k, first[k])

            def group(g, idvec):
                nxt = idv[pl.ds(off + jnp.minimum((g + 1) * L, H - L), L)]
                for k in range(NBUF):
                    drain(tab, k)
                    extract(k, idvec[k], rows, g * L + k)
                    fire(tab, k, idvec[k + NBUF])
                for k in range(NBUF, L):
                    s = k - NBUF
                    drain(tab, s)
                    extract(s, idvec[k], rows, g * L + k)

                    @pl.when(g < ng - 1)
                    def _():
                        fire(tab, s, nxt[s])
                return nxt

            lax.fori_loop(0, ng, group, first)

        def dot_phase(half):
            def group(g, carry):
                rbase = (g * L + iota) * D
                acc = jnp.zeros((L,), jnp.float32)
                for d in range(D):
                    u = plsc.load_gather(urows, [rbase + d])
                    v = plsc.load_gather(irows, [rbase + d])
                    acc = acc + u * v
                outv[pl.ds(half * H + g * L, L)] = acc
                return carry

            lax.fori_loop(0, ng, group, 0)

        for half in range(n_half):
            fetch_phase(ut_hbm, uidv, urows, half)
            fetch_phase(it_hbm, iidv, irows, half)
            dot_phase(half)

        pltpu.sync_copy(outv, out_hbm.at[pl.ds(base, b_per_w)])

    return bprmf


def kernel(u_ids, i_ids, user_table, item_table):
    return _build()(u_ids.astype(jnp.int32), i_ids.astype(jnp.int32),
                    user_table.T, item_table.T)
